# Initial kernel scaffold; baseline (speedup 1.0000x reference)
import jax
import jax.numpy as jnp
from jax.experimental import pallas as pl


def kernel(pos, edge_index, W_l1, b_l1, W_r1, W_se1, W_l2, b_l2, W_r2, W_se2, W_m3, b_m3, W_m4, b_m4, alpha):
    raise NotImplementedError("write your pallas kernel here")


# speedup vs baseline: 10.3419x; 10.3419x over previous
import functools

import jax
import jax.numpy as jnp
from jax import lax
from jax.experimental import pallas as pl
from jax.experimental.pallas import tpu as pltpu
from jax.experimental.pallas import tpu_sc as plsc

N = 100000
E = 1600000
H = 32
HH = H // 2

NC = 2
NS = 16
NW = NC * NS

KA = 6256
TA = 8 * KA
EA = TA * NW
NA = NS * KA

TC_E = E // NS
KC = 1000
NP = N // NS

_LRELU = 0.01
_INV_SQRT_H = 1.0 / (H ** 0.5)


def _seg_sum_l1(x2p, srcp, dstp, zeros_a):
    mesh = plsc.VectorSubcoreMesh(core_axis_name="c", subcore_axis_name="s")

    @functools.partial(
        pl.kernel,
        out_type=jax.ShapeDtypeStruct((NC * NA, 8), jnp.float32),
        mesh=mesh,
        scratch_types=[
            pltpu.VMEM((KA,), jnp.int32),
            pltpu.VMEM((KA,), jnp.int32),
            pltpu.VMEM((KA, 8), jnp.float32),
            pltpu.VMEM_SHARED((NA, 8), jnp.float32),
            pltpu.SemaphoreType.DMA,
        ],
        compiler_params=pltpu.CompilerParams(use_tc_tiling_on_sc=False),
    )
    def k(x2_hbm, src_hbm, dst_hbm, z_hbm, out_hbm, idx_v, dst_v, rows_v,
          accum, sem):
        c = lax.axis_index("c")
        s = lax.axis_index("s")
        wid = s * NC + c
        pltpu.sync_copy(z_hbm.at[pl.ds(s * KA, KA), :],
                        accum.at[pl.ds(s * KA, KA), :])
        plsc.subcore_barrier()
        base = wid * TA

        def body(j, carry):
            off = base + j * KA
            pltpu.sync_copy(src_hbm.at[pl.ds(off, KA)], idx_v)
            pltpu.async_copy(x2_hbm.at[idx_v], rows_v, sem).wait()
            pltpu.sync_copy(dst_hbm.at[pl.ds(off, KA)], dst_v)
            pltpu.sync_copy(rows_v, accum.at[dst_v], add=True)
            return carry

        lax.fori_loop(0, TA // KA, body, 0)
        plsc.subcore_barrier()
        pltpu.sync_copy(accum.at[pl.ds(s * KA, KA), :],
                        out_hbm.at[pl.ds(c * NA + s * KA, KA), :])

    return k(x2p, srcp, dstp, zeros_a)


def _seg_sum_l2(h1s, src2, dst, zeros_c):
    mesh = plsc.VectorSubcoreMesh(core_axis_name="c", subcore_axis_name="s")

    @functools.partial(
        pl.kernel,
        out_type=jax.ShapeDtypeStruct((NC * N, HH), jnp.float32),
        mesh=mesh,
        scratch_types=[
            pltpu.VMEM((KC,), jnp.int32),
            pltpu.VMEM((KC,), jnp.int32),
            pltpu.VMEM((KC, HH), jnp.float32),
            pltpu.VMEM_SHARED((N, HH), jnp.float32),
            pltpu.SemaphoreType.DMA,
        ],
        compiler_params=pltpu.CompilerParams(use_tc_tiling_on_sc=False),
    )
    def k(h1_hbm, src_hbm, dst_hbm, z_hbm, out_hbm, idx_v, dst_v, rows_v,
          accum, sem):
        c = lax.axis_index("c")
        s = lax.axis_index("s")
        pltpu.sync_copy(z_hbm.at[pl.ds(s * NP, NP), :],
                        accum.at[pl.ds(s * NP, NP), :])
        plsc.subcore_barrier()
        ebase = s * TC_E
        sbase = c * E + ebase

        def body(j, carry):
            pltpu.sync_copy(src_hbm.at[pl.ds(sbase + j * KC, KC)], idx_v)
            pltpu.async_copy(h1_hbm.at[idx_v], rows_v, sem).wait()
            pltpu.sync_copy(dst_hbm.at[pl.ds(ebase + j * KC, KC)], dst_v)
            pltpu.sync_copy(rows_v, accum.at[dst_v], add=True)
            return carry

        lax.fori_loop(0, TC_E // KC, body, 0)
        plsc.subcore_barrier()
        pltpu.sync_copy(accum.at[pl.ds(s * NP, NP), :],
                        out_hbm.at[pl.ds(c * N + s * NP, NP), :])

    return k(h1s, src2, dst, zeros_c)


BB = 2000


def _tc_h1(aggp, x2, W_l1, b_l1, W_r1, W_se1):
    def body(ap_ref, x_ref, wl_ref, bl_ref, wr_ref, wse_ref, out_ref):
        agg = ap_ref[0] + ap_ref[1]
        x = x_ref[...]
        z = (agg[:, 0:1] * wl_ref[0:1, :] + agg[:, 1:2] * wl_ref[1:2, :]
             + x[:, 0:1] * wr_ref[0:1, :] + x[:, 1:2] * wr_ref[1:2, :]
             + bl_ref[...])
        z = jnp.where(z > 0, z, _LRELU * z)
        h1 = jnp.dot(z, wse_ref[...],
                     preferred_element_type=jnp.float32, precision=lax.Precision.HIGHEST) * _INV_SQRT_H
        out_ref[0, :, :] = h1[:, :HH]
        out_ref[1, :, :] = h1[:, HH:]

    return pl.pallas_call(
        body,
        grid=(N // BB,),
        in_specs=[
            pl.BlockSpec((2, BB, 2), lambda i: (0, i, 0)),
            pl.BlockSpec((BB, 2), lambda i: (i, 0)),
            pl.BlockSpec((2, H), lambda i: (0, 0)),
            pl.BlockSpec((1, H), lambda i: (0, 0)),
            pl.BlockSpec((2, H), lambda i: (0, 0)),
            pl.BlockSpec((H, H), lambda i: (0, 0)),
        ],
        out_specs=pl.BlockSpec((2, BB, HH), lambda i: (0, i, 0)),
        out_shape=jax.ShapeDtypeStruct((2, N, HH), jnp.float32),
    )(aggp, x2, W_l1, b_l1, W_r1, W_se1)


def _tc_head(agg2s, h1s, W_l2, b_l2, W_r2, W_se2, W_m3, b_m3, W_m4, b_m4,
             alpha):
    def body(a0_ref, a1_ref, h0_ref, h1_ref, wl_ref, bl_ref, wr_ref,
             wse_ref, wm3_ref, bm3_ref, wm4_ref, bm4_ref, al_ref, out_ref):
        agg2 = jnp.concatenate([a0_ref[...], a1_ref[...]], axis=1)
        h1 = jnp.concatenate([h0_ref[...], h1_ref[...]], axis=1)
        z = (jnp.dot(agg2, wl_ref[...], preferred_element_type=jnp.float32, precision=lax.Precision.HIGHEST)
             + jnp.dot(h1, wr_ref[...], preferred_element_type=jnp.float32, precision=lax.Precision.HIGHEST)
             + bl_ref[...])
        z = jnp.where(z > 0, z, _LRELU * z)
        h2 = jnp.dot(z, wse_ref[...],
                     preferred_element_type=jnp.float32, precision=lax.Precision.HIGHEST) * _INV_SQRT_H
        skip = al_ref[0, 0] * h1 + h2
        o = jnp.dot(skip, wm3_ref[...], preferred_element_type=jnp.float32, precision=lax.Precision.HIGHEST)
        o = jnp.maximum(o + bm3_ref[...], 0.0)
        out_ref[...] = (jnp.dot(o, wm4_ref[...],
                                preferred_element_type=jnp.float32, precision=lax.Precision.HIGHEST)
                        + bm4_ref[...])

    nb = N // BB
    return pl.pallas_call(
        body,
        grid=(nb,),
        in_specs=[
            pl.BlockSpec((BB, HH), lambda i: (i, 0)),
            pl.BlockSpec((BB, HH), lambda i: (nb + i, 0)),
            pl.BlockSpec((BB, HH), lambda i: (i, 0)),
            pl.BlockSpec((BB, HH), lambda i: (nb + i, 0)),
            pl.BlockSpec((H, H), lambda i: (0, 0)),
            pl.BlockSpec((1, H), lambda i: (0, 0)),
            pl.BlockSpec((H, H), lambda i: (0, 0)),
            pl.BlockSpec((H, H), lambda i: (0, 0)),
            pl.BlockSpec((H, H), lambda i: (0, 0)),
            pl.BlockSpec((1, H), lambda i: (0, 0)),
            pl.BlockSpec((H, 1), lambda i: (0, 0)),
            pl.BlockSpec((1, 1), lambda i: (0, 0)),
            pl.BlockSpec(memory_space=pltpu.SMEM),
        ],
        out_specs=pl.BlockSpec((BB, 1), lambda i: (i, 0)),
        out_shape=jax.ShapeDtypeStruct((N, 1), jnp.float32),
    )(agg2s, agg2s, h1s, h1s, W_l2, b_l2, W_r2, W_se2, W_m3, b_m3, W_m4,
      b_m4, alpha)


def kernel(pos, edge_index, W_l1, b_l1, W_r1, W_se1, W_l2, b_l2, W_r2,
           W_se2, W_m3, b_m3, W_m4, b_m4, alpha):
    src = edge_index[0]
    dst = edge_index[1]
    x2 = pos[:, :2]

    pad = EA - E
    srcp = jnp.concatenate([src, jnp.full((pad,), N, jnp.int32)])
    dstp = jnp.concatenate([dst, jnp.full((pad,), N, jnp.int32)])
    x2p = jnp.pad(x2, ((0, 8), (0, 6)))
    src2 = jnp.concatenate([src, src + N])
    zeros_a = jnp.zeros((NA, 8), jnp.float32)
    zeros_c = jnp.zeros((N, HH), jnp.float32)

    aggp = _seg_sum_l1(x2p, srcp, dstp, zeros_a)
    aggp = aggp.reshape(NC, NA, 8)[:, :N, :2]

    h1_halves = _tc_h1(aggp, x2, W_l1, b_l1.reshape(1, H), W_r1, W_se1)
    h1s = h1_halves.reshape(2 * N, HH)

    agg2s = _seg_sum_l2(h1s, src2, dst, zeros_c)

    pred = _tc_head(agg2s, h1s, W_l2, b_l2.reshape(1, H), W_r2, W_se2,
                    W_m3, b_m3.reshape(1, H), W_m4, b_m4.reshape(1, 1),
                    jnp.asarray(alpha, jnp.float32).reshape(1, 1))
    return pred[:, 0]


# speedup vs baseline: 10.7944x; 1.0438x over previous
import functools

import jax
import jax.numpy as jnp
from jax import lax
from jax.experimental import pallas as pl
from jax.experimental.pallas import tpu as pltpu
from jax.experimental.pallas import tpu_sc as plsc

N = 100000
E = 1600000
H = 32
HH = H // 2

NC = 2
NS = 16
NW = NC * NS
NP = N // NS

KA = 6256
TA = 8 * KA
EA = TA * NW

TC_E = E // NS
KC = 800

_LRELU = 0.01
_INV_SQRT_H = 1.0 / (H ** 0.5)
_P = lax.Precision.HIGHEST


def _seg_sum_l1(x8, srcp, dstp, zeros_a):
    mesh = plsc.VectorSubcoreMesh(core_axis_name="c", subcore_axis_name="s")

    @functools.partial(
        pl.kernel,
        out_type=jax.ShapeDtypeStruct((NC * N, 8), jnp.float32),
        mesh=mesh,
        scratch_types=[
            pltpu.VMEM((KA,), jnp.int32),
            pltpu.VMEM((KA,), jnp.int32),
            pltpu.VMEM((KA, 8), jnp.float32),
            pltpu.VMEM_SHARED((N + 8, 8), jnp.float32),
            pltpu.SemaphoreType.DMA,
        ],
        compiler_params=pltpu.CompilerParams(use_tc_tiling_on_sc=False),
    )
    def k(x8_hbm, src_hbm, dst_hbm, z_hbm, out_hbm, idx_v, dst_v, rows_v,
          accum, sem):
        c = lax.axis_index("c")
        s = lax.axis_index("s")
        wid = s * NC + c
        pltpu.sync_copy(z_hbm.at[pl.ds(s * NP, NP), :],
                        accum.at[pl.ds(s * NP, NP), :])
        plsc.subcore_barrier()
        base = wid * TA

        def body(j, carry):
            off = base + j * KA
            pltpu.sync_copy(src_hbm.at[pl.ds(off, KA)], idx_v)
            pltpu.async_copy(x8_hbm.at[idx_v], rows_v, sem).wait()
            pltpu.sync_copy(dst_hbm.at[pl.ds(off, KA)], dst_v)
            pltpu.sync_copy(rows_v, accum.at[dst_v], add=True)
            return carry

        lax.fori_loop(0, TA // KA, body, 0)
        plsc.subcore_barrier()
        pltpu.sync_copy(accum.at[pl.ds(s * NP, NP), :],
                        out_hbm.at[pl.ds(c * N + s * NP, NP), :])

    return k(x8, srcp, dstp, zeros_a)


def _seg_sum_l2(h1s, src, dst, zeros_c):
    mesh = plsc.VectorSubcoreMesh(core_axis_name="c", subcore_axis_name="s")

    @functools.partial(
        pl.kernel,
        out_type=jax.ShapeDtypeStruct((NC * N, HH), jnp.float32),
        mesh=mesh,
        scratch_types=[
            pltpu.VMEM((KC,), jnp.int32),
            pltpu.VMEM((KC,), jnp.int32),
            pltpu.VMEM((KC, HH), jnp.float32),
            pltpu.VMEM_SHARED((N, HH), jnp.float32),
            pltpu.SemaphoreType.DMA,
        ],
        compiler_params=pltpu.CompilerParams(use_tc_tiling_on_sc=False),
    )
    def k(h1_hbm, src_hbm, dst_hbm, z_hbm, out_hbm, idx_v, dst_v, rows_v,
          accum, sem):
        c = lax.axis_index("c")
        s = lax.axis_index("s")
        pltpu.sync_copy(z_hbm.at[pl.ds(s * NP, NP), :],
                        accum.at[pl.ds(s * NP, NP), :])
        plsc.subcore_barrier()
        ebase = s * TC_E
        cN = jnp.broadcast_to(c * N, (16,))

        def body(j, carry):
            off = ebase + j * KC
            pltpu.sync_copy(src_hbm.at[pl.ds(off, KC)], idx_v)

            def add_off(i, carry2):
                idx_v[pl.ds(i * 16, 16)] = idx_v[pl.ds(i * 16, 16)] + cN
                return carry2

            lax.fori_loop(0, KC // 16, add_off, 0)
            pltpu.async_copy(h1_hbm.at[idx_v], rows_v, sem).wait()
            pltpu.sync_copy(dst_hbm.at[pl.ds(off, KC)], dst_v)
            pltpu.sync_copy(rows_v, accum.at[dst_v], add=True)
            return carry

        lax.fori_loop(0, TC_E // KC, body, 0)
        plsc.subcore_barrier()
        pltpu.sync_copy(accum.at[pl.ds(s * NP, NP), :],
                        out_hbm.at[pl.ds(c * N + s * NP, NP), :])

    return k(h1s, src, dst, zeros_c)


BB = 4000
NB = N // BB


def _tc_h1(aggp, pos, W_l1, b_l1, W_r1, W_se1):
    def body(a0_ref, a1_ref, x_ref, wl_ref, bl_ref, wr_ref, wse_ref,
             out_ref):
        agg = a0_ref[:, :2] + a1_ref[:, :2]
        x = x_ref[:, :2]
        z = (agg[:, 0:1] * wl_ref[0:1, :] + agg[:, 1:2] * wl_ref[1:2, :]
             + x[:, 0:1] * wr_ref[0:1, :] + x[:, 1:2] * wr_ref[1:2, :]
             + bl_ref[...])
        z = jnp.where(z > 0, z, _LRELU * z)
        out_ref[...] = jnp.dot(z, wse_ref[0],
                               preferred_element_type=jnp.float32,
                               precision=_P) * _INV_SQRT_H

    return pl.pallas_call(
        body,
        grid=(2, NB),
        in_specs=[
            pl.BlockSpec((BB, 8), lambda h, i: (i, 0)),
            pl.BlockSpec((BB, 8), lambda h, i: (NB + i, 0)),
            pl.BlockSpec((BB, 3), lambda h, i: (i, 0)),
            pl.BlockSpec((2, H), lambda h, i: (0, 0)),
            pl.BlockSpec((1, H), lambda h, i: (0, 0)),
            pl.BlockSpec((2, H), lambda h, i: (0, 0)),
            pl.BlockSpec((1, H, HH), lambda h, i: (h, 0, 0)),
        ],
        out_specs=pl.BlockSpec((BB, HH), lambda h, i: (h * NB + i, 0)),
        out_shape=jax.ShapeDtypeStruct((2 * N, HH), jnp.float32),
    )(aggp, aggp, pos, W_l1, b_l1, W_r1, W_se1)


def _tc_head(agg2s, h1s, Wcat, b_l2, W_se2, W_m3, b_m3, W_m4, b_m4,
             alpha):
    def body(a0_ref, a1_ref, h0_ref, h1_ref, wcat_ref, bl_ref, wse_ref,
             wm3_ref, bm3_ref, wm4_ref, bm4_ref, al_ref, out_ref):
        zcat = jnp.concatenate(
            [a0_ref[...], a1_ref[...], h0_ref[...], h1_ref[...]], axis=1)
        h1 = zcat[:, H:]
        z = jnp.dot(zcat, wcat_ref[...], preferred_element_type=jnp.float32,
                    precision=_P) + bl_ref[...]
        z = jnp.where(z > 0, z, _LRELU * z)
        m = jnp.dot(wse_ref[...], wm3_ref[...],
                    preferred_element_type=jnp.float32,
                    precision=_P) * _INV_SQRT_H
        o = (al_ref[0, 0]
             * jnp.dot(h1, wm3_ref[...], preferred_element_type=jnp.float32,
                       precision=_P)
             + jnp.dot(z, m, preferred_element_type=jnp.float32,
                       precision=_P)
             + bm3_ref[...])
        o = jnp.maximum(o, 0.0)
        out_ref[...] = (jnp.dot(o, wm4_ref[...],
                                preferred_element_type=jnp.float32,
                                precision=_P)
                        + bm4_ref[...])

    return pl.pallas_call(
        body,
        grid=(NB,),
        in_specs=[
            pl.BlockSpec((BB, HH), lambda i: (i, 0)),
            pl.BlockSpec((BB, HH), lambda i: (NB + i, 0)),
            pl.BlockSpec((BB, HH), lambda i: (i, 0)),
            pl.BlockSpec((BB, HH), lambda i: (NB + i, 0)),
            pl.BlockSpec((2 * H, H), lambda i: (0, 0)),
            pl.BlockSpec((1, H), lambda i: (0, 0)),
            pl.BlockSpec((H, H), lambda i: (0, 0)),
            pl.BlockSpec((H, H), lambda i: (0, 0)),
            pl.BlockSpec((1, H), lambda i: (0, 0)),
            pl.BlockSpec((H, 1), lambda i: (0, 0)),
            pl.BlockSpec((1, 1), lambda i: (0, 0)),
            pl.BlockSpec((1, 1), lambda i: (0, 0), memory_space=pltpu.SMEM),
        ],
        out_specs=pl.BlockSpec((BB, 1), lambda i: (i, 0)),
        out_shape=jax.ShapeDtypeStruct((N, 1), jnp.float32),
    )(agg2s, agg2s, h1s, h1s, Wcat, b_l2, W_se2, W_m3, b_m3, W_m4,
      b_m4, alpha)


def kernel(pos, edge_index, W_l1, b_l1, W_r1, W_se1, W_l2, b_l2, W_r2,
           W_se2, W_m3, b_m3, W_m4, b_m4, alpha):
    eip = jnp.pad(edge_index, ((0, 0), (0, EA - E)), constant_values=N)
    srcp, dstp = eip[0], eip[1]
    src, dst = edge_index[0], edge_index[1]
    x8 = jnp.pad(pos, ((0, 8), (0, 5)))
    zeros_a = jnp.zeros((N, 8), jnp.float32)
    zeros_c = jnp.zeros((N, HH), jnp.float32)

    aggp = _seg_sum_l1(x8, srcp, dstp, zeros_a)

    wse1h = W_se1.reshape(H, 2, HH).transpose(1, 0, 2)
    h1s = _tc_h1(aggp, pos, W_l1, b_l1.reshape(1, H), W_r1, wse1h)

    agg2s = _seg_sum_l2(h1s, src, dst, zeros_c)

    Wcat = jnp.concatenate([W_l2, W_r2], axis=0)
    pred = _tc_head(agg2s, h1s, Wcat, b_l2.reshape(1, H), W_se2,
                    W_m3, b_m3.reshape(1, H), W_m4, b_m4.reshape(1, 1),
                    jnp.asarray(alpha, jnp.float32).reshape(1, 1))
    return pred[:, 0]


# speedup vs baseline: 12.4509x; 1.1535x over previous
import functools

import jax
import jax.numpy as jnp
from jax import lax
from jax.experimental import pallas as pl
from jax.experimental.pallas import tpu as pltpu
from jax.experimental.pallas import tpu_sc as plsc

N = 100000
E = 1600000
H = 32
HH = H // 2

NC = 2
NS = 16
NW = NC * NS
NP = N // NS

KA = 6256
TA = 8 * KA
EA = TA * NW

TC_E = E // NS
KC = 800

_LRELU = 0.01
_INV_SQRT_H = 1.0 / (H ** 0.5)
_P = lax.Precision.HIGHEST


def _seg_sum_l1(x8, srcp, dstp, zeros_a):
    mesh = plsc.VectorSubcoreMesh(core_axis_name="c", subcore_axis_name="s")

    @functools.partial(
        pl.kernel,
        out_type=jax.ShapeDtypeStruct((NC * N, 8), jnp.float32),
        mesh=mesh,
        scratch_types=[
            pltpu.VMEM((KA,), jnp.int32),
            pltpu.VMEM((KA,), jnp.int32),
            pltpu.VMEM((KA, 8), jnp.float32),
            pltpu.VMEM_SHARED((N + 8, 8), jnp.float32),
            pltpu.SemaphoreType.DMA,
        ],
        compiler_params=pltpu.CompilerParams(use_tc_tiling_on_sc=False),
    )
    def k(x8_hbm, src_hbm, dst_hbm, z_hbm, out_hbm, idx_v, dst_v, rows_v,
          accum, sem):
        c = lax.axis_index("c")
        s = lax.axis_index("s")
        wid = s * NC + c
        pltpu.sync_copy(z_hbm.at[pl.ds(s * NP, NP), :],
                        accum.at[pl.ds(s * NP, NP), :])
        plsc.subcore_barrier()
        base = wid * TA

        def body(j, carry):
            off = base + j * KA
            pltpu.sync_copy(src_hbm.at[pl.ds(off, KA)], idx_v)
            pltpu.async_copy(x8_hbm.at[idx_v], rows_v, sem).wait()
            pltpu.sync_copy(dst_hbm.at[pl.ds(off, KA)], dst_v)
            pltpu.sync_copy(rows_v, accum.at[dst_v], add=True)
            return carry

        lax.fori_loop(0, TA // KA, body, 0)
        plsc.subcore_barrier()
        pltpu.sync_copy(accum.at[pl.ds(s * NP, NP), :],
                        out_hbm.at[pl.ds(c * N + s * NP, NP), :])

    return k(x8, srcp, dstp, zeros_a)


def _seg_sum_l2(h1s, src, dst, zeros_c):
    mesh = plsc.VectorSubcoreMesh(core_axis_name="c", subcore_axis_name="s")

    @functools.partial(
        pl.kernel,
        out_type=jax.ShapeDtypeStruct((NC * N, HH), jnp.float32),
        mesh=mesh,
        scratch_types=[
            pltpu.VMEM((KC,), jnp.int32),
            pltpu.VMEM((KC,), jnp.int32),
            pltpu.VMEM((KC,), jnp.int32),
            pltpu.VMEM((KC,), jnp.int32),
            pltpu.VMEM((KC, HH), jnp.float32),
            pltpu.VMEM((KC, HH), jnp.float32),
            pltpu.SemaphoreType.DMA,
            pltpu.SemaphoreType.DMA,
            pltpu.VMEM_SHARED((N, HH), jnp.float32),
        ],
        compiler_params=pltpu.CompilerParams(use_tc_tiling_on_sc=False),
    )
    def k(h1_hbm, src_hbm, dst_hbm, z_hbm, out_hbm, idx0, idx1, dst0, dst1,
          rows0, rows1, sem0, sem1, accum):
        c = lax.axis_index("c")
        s = lax.axis_index("s")
        pltpu.sync_copy(z_hbm.at[pl.ds(s * NP, NP), :],
                        accum.at[pl.ds(s * NP, NP), :])
        plsc.subcore_barrier()
        ebase = s * TC_E
        cN = jnp.broadcast_to(c * N, (16,))
        idx = (idx0, idx1)
        dstb = (dst0, dst1)
        rows = (rows0, rows1)
        sems = (sem0, sem1)
        nch = TC_E // KC

        def load_and_fire(j, b):
            off = ebase + j * KC
            pltpu.sync_copy(src_hbm.at[pl.ds(off, KC)], idx[b])

            def add_off(i, carry2):
                idx[b][pl.ds(i * 16, 16)] = idx[b][pl.ds(i * 16, 16)] + cN
                return carry2

            lax.fori_loop(0, KC // 16, add_off, 0)
            pltpu.sync_copy(dst_hbm.at[pl.ds(off, KC)], dstb[b])
            pltpu.async_copy(h1_hbm.at[idx[b]], rows[b], sems[b])

        def drain_and_add(b):
            pltpu.make_async_copy(h1_hbm.at[idx[b]], rows[b], sems[b]).wait()
            pltpu.sync_copy(rows[b], accum.at[dstb[b]], add=True)

        load_and_fire(0, 0)

        def outer(k2, carry):
            for b in (0, 1):
                j = 2 * k2 + b
                load_and_fire(j + 1, 1 - b)
                drain_and_add(b)
            return carry

        lax.fori_loop(0, (nch - 1) // 2, outer, 0)
        drain_and_add(0)
        plsc.subcore_barrier()
        pltpu.sync_copy(accum.at[pl.ds(s * NP, NP), :],
                        out_hbm.at[pl.ds(c * N + s * NP, NP), :])

    return k(h1s, src, dst, zeros_c)


BB = 4000
NB = N // BB


def _tc_h1(aggp, pos, W_l1, b_l1, W_r1, W_se1):
    def body(a0_ref, a1_ref, x_ref, wl_ref, bl_ref, wr_ref, wse_ref,
             out_ref):
        agg = a0_ref[:, :2] + a1_ref[:, :2]
        x = x_ref[:, :2]
        z = (agg[:, 0:1] * wl_ref[0:1, :] + agg[:, 1:2] * wl_ref[1:2, :]
             + x[:, 0:1] * wr_ref[0:1, :] + x[:, 1:2] * wr_ref[1:2, :]
             + bl_ref[...])
        z = jnp.where(z > 0, z, _LRELU * z)
        out_ref[...] = jnp.dot(z, wse_ref[0],
                               preferred_element_type=jnp.float32,
                               precision=_P) * _INV_SQRT_H

    return pl.pallas_call(
        body,
        grid=(2, NB),
        in_specs=[
            pl.BlockSpec((BB, 8), lambda h, i: (i, 0)),
            pl.BlockSpec((BB, 8), lambda h, i: (NB + i, 0)),
            pl.BlockSpec((BB, 3), lambda h, i: (i, 0)),
            pl.BlockSpec((2, H), lambda h, i: (0, 0)),
            pl.BlockSpec((1, H), lambda h, i: (0, 0)),
            pl.BlockSpec((2, H), lambda h, i: (0, 0)),
            pl.BlockSpec((1, H, HH), lambda h, i: (h, 0, 0)),
        ],
        out_specs=pl.BlockSpec((BB, HH), lambda h, i: (h * NB + i, 0)),
        out_shape=jax.ShapeDtypeStruct((2 * N, HH), jnp.float32),
    )(aggp, aggp, pos, W_l1, b_l1, W_r1, W_se1)


def _tc_head(agg2s, h1s, Wcat, b_l2, W_se2, W_m3, b_m3, W_m4, b_m4,
             alpha):
    def body(a0_ref, a1_ref, h0_ref, h1_ref, wcat_ref, bl_ref, wse_ref,
             wm3_ref, bm3_ref, wm4_ref, bm4_ref, al_ref, out_ref):
        zcat = jnp.concatenate(
            [a0_ref[...], a1_ref[...], h0_ref[...], h1_ref[...]], axis=1)
        h1 = zcat[:, H:]
        z = jnp.dot(zcat, wcat_ref[...], preferred_element_type=jnp.float32,
                    precision=_P) + bl_ref[...]
        z = jnp.where(z > 0, z, _LRELU * z)
        m = jnp.dot(wse_ref[...], wm3_ref[...],
                    preferred_element_type=jnp.float32,
                    precision=_P) * _INV_SQRT_H
        o = (al_ref[0, 0]
             * jnp.dot(h1, wm3_ref[...], preferred_element_type=jnp.float32,
                       precision=_P)
             + jnp.dot(z, m, preferred_element_type=jnp.float32,
                       precision=_P)
             + bm3_ref[...])
        o = jnp.maximum(o, 0.0)
        out_ref[...] = (jnp.dot(o, wm4_ref[...],
                                preferred_element_type=jnp.float32,
                                precision=_P)
                        + bm4_ref[...])

    return pl.pallas_call(
        body,
        grid=(NB,),
        in_specs=[
            pl.BlockSpec((BB, HH), lambda i: (i, 0)),
            pl.BlockSpec((BB, HH), lambda i: (NB + i, 0)),
            pl.BlockSpec((BB, HH), lambda i: (i, 0)),
            pl.BlockSpec((BB, HH), lambda i: (NB + i, 0)),
            pl.BlockSpec((2 * H, H), lambda i: (0, 0)),
            pl.BlockSpec((1, H), lambda i: (0, 0)),
            pl.BlockSpec((H, H), lambda i: (0, 0)),
            pl.BlockSpec((H, H), lambda i: (0, 0)),
            pl.BlockSpec((1, H), lambda i: (0, 0)),
            pl.BlockSpec((H, 1), lambda i: (0, 0)),
            pl.BlockSpec((1, 1), lambda i: (0, 0)),
            pl.BlockSpec((1, 1), lambda i: (0, 0), memory_space=pltpu.SMEM),
        ],
        out_specs=pl.BlockSpec((BB, 1), lambda i: (i, 0)),
        out_shape=jax.ShapeDtypeStruct((N, 1), jnp.float32),
    )(agg2s, agg2s, h1s, h1s, Wcat, b_l2, W_se2, W_m3, b_m3, W_m4,
      b_m4, alpha)


def kernel(pos, edge_index, W_l1, b_l1, W_r1, W_se1, W_l2, b_l2, W_r2,
           W_se2, W_m3, b_m3, W_m4, b_m4, alpha):
    eip = jnp.pad(edge_index, ((0, 0), (0, EA - E)), constant_values=N)
    srcp, dstp = eip[0], eip[1]
    src, dst = edge_index[0], edge_index[1]
    x8 = jnp.pad(pos, ((0, 8), (0, 5)))
    zeros_a = jnp.zeros((N, 8), jnp.float32)
    zeros_c = jnp.zeros((N, HH), jnp.float32)

    aggp = _seg_sum_l1(x8, srcp, dstp, zeros_a)

    wse1h = W_se1.reshape(H, 2, HH).transpose(1, 0, 2)
    h1s = _tc_h1(aggp, pos, W_l1, b_l1.reshape(1, H), W_r1, wse1h)

    agg2s = _seg_sum_l2(h1s, src, dst, zeros_c)

    Wcat = jnp.concatenate([W_l2, W_r2], axis=0)
    pred = _tc_head(agg2s, h1s, Wcat, b_l2.reshape(1, H), W_se2,
                    W_m3, b_m3.reshape(1, H), W_m4, b_m4.reshape(1, 1),
                    jnp.asarray(alpha, jnp.float32).reshape(1, 1))
    return pred[:, 0]
